# Initial kernel scaffold; baseline (speedup 1.0000x reference)
import jax
import jax.numpy as jnp
from jax.experimental import pallas as pl


def kernel(x, edge_index, edge_attr, sage_Wl, sage_bl, sage_Wr, Wq, bq, Wk, bk, Wv, bv, We, be, Wskip, bskip, Wg, att_src, att_dst, gat_b, fuse_W, fuse_b, ln1_w, ln1_b, ln2_w, ln2_b, gate_s, gate_a, gate_n):
    raise NotImplementedError("write your pallas kernel here")


# speedup vs baseline: 16.8143x; 16.8143x over previous
import functools

import jax
import jax.numpy as jnp
from jax import lax
from jax.experimental import pallas as pl
from jax.experimental.pallas import tpu as pltpu
from jax.experimental.pallas import tpu_sc as plsc

H = 4
ED = 16

_SC_PARAMS = dict(
    compiler_params=pltpu.CompilerParams(
        needs_layout_passes=False, use_tc_tiling_on_sc=False),
)


def _prep_body(x_ref, WqT_ref, bq_ref, WkT_ref, bk_ref, WvT_ref, bv_ref,
               We_ref, WgT_ref, asrc_ref, adst_ref,
               q_ref, k_ref, v_ref, xh_ref, qe_ref, asd_ref):
    x = x_ref[...]
    q = jnp.dot(x, WqT_ref[...], preferred_element_type=jnp.float32) + bq_ref[...]
    k = jnp.dot(x, WkT_ref[...], preferred_element_type=jnp.float32) + bk_ref[...]
    v = jnp.dot(x, WvT_ref[...], preferred_element_type=jnp.float32) + bv_ref[...]
    xh = jnp.dot(x, WgT_ref[...], preferred_element_type=jnp.float32)
    q_ref[...] = q
    k_ref[...] = k
    v_ref[...] = v
    xh_ref[...] = xh
    D = x.shape[1]
    qe_parts = []
    for h in range(H):
        qe_parts.append(jnp.dot(q[:, h * D:(h + 1) * D],
                                We_ref[...][h * D:(h + 1) * D, :],
                                preferred_element_type=jnp.float32))
    qe_ref[...] = jnp.concatenate(qe_parts, axis=1)
    asd_parts = []
    for h in range(H):
        asd_parts.append(jnp.sum(xh[:, h * D:(h + 1) * D] * asrc_ref[...][h:h + 1, :],
                                 axis=1, keepdims=True))
    for h in range(H):
        asd_parts.append(jnp.sum(xh[:, h * D:(h + 1) * D] * adst_ref[...][h:h + 1, :],
                                 axis=1, keepdims=True))
    asd_ref[...] = jnp.concatenate(asd_parts, axis=1)


def _tc_prep(x, WqT, bq, WkT, bk, WvT, bv, We, WgT, att_src, att_dst):
    N, D = x.shape
    HD = H * D
    BN = 1000
    grid = (N // BN,)
    full = lambda shape: pl.BlockSpec(shape, lambda i: tuple(0 for _ in shape))
    return pl.pallas_call(
        _prep_body,
        grid=grid,
        in_specs=[
            pl.BlockSpec((BN, D), lambda i: (i, 0)),
            full((D, HD)), full((1, HD)),
            full((D, HD)), full((1, HD)),
            full((D, HD)), full((1, HD)),
            full((HD, ED)),
            full((D, HD)),
            full((H, D)), full((H, D)),
        ],
        out_specs=[
            pl.BlockSpec((BN, HD), lambda i: (i, 0)),
            pl.BlockSpec((BN, HD), lambda i: (i, 0)),
            pl.BlockSpec((BN, HD), lambda i: (i, 0)),
            pl.BlockSpec((BN, HD), lambda i: (i, 0)),
            pl.BlockSpec((BN, H * ED), lambda i: (i, 0)),
            pl.BlockSpec((BN, 2 * H), lambda i: (i, 0)),
        ],
        out_shape=[
            jax.ShapeDtypeStruct((N, HD), jnp.float32),
            jax.ShapeDtypeStruct((N, HD), jnp.float32),
            jax.ShapeDtypeStruct((N, HD), jnp.float32),
            jax.ShapeDtypeStruct((N, HD), jnp.float32),
            jax.ShapeDtypeStruct((N, H * ED), jnp.float32),
            jax.ShapeDtypeStruct((N, 2 * H), jnp.float32),
        ],
    )(x, WqT, bq, WkT, bk, WvT, bv, We, WgT, att_src, att_dst)


_CA = 40


def _sc_pass_a1(QT, KT, src, dst, ea, zero_sd):
    N = QT.shape[0]
    NP = zero_sd.shape[0]
    E = src.shape[0]
    per_tile = E // 32
    n_chunks = per_tile // _CA
    rows_per_tile = NP // 16
    SCALE = 1.0 / (128.0 ** 0.5)
    mesh = plsc.VectorSubcoreMesh(core_axis_name="c", subcore_axis_name="s")

    @functools.partial(
        pl.kernel, mesh=mesh,
        out_type=[
            jax.ShapeDtypeStruct((E, 16), jnp.float32),
            jax.ShapeDtypeStruct((2, NP, 16), jnp.float32),
        ],
        scratch_types=[
            pltpu.VMEM((_CA,), jnp.int32),
            pltpu.VMEM((_CA,), jnp.int32),
            pltpu.VMEM((_CA, 592), jnp.float32),
            pltpu.VMEM((_CA, 528), jnp.float32),
            pltpu.VMEM((_CA, 16), jnp.float32),
            pltpu.VMEM((_CA, 16), jnp.float32),
            pltpu.VMEM_SHARED((NP, 16), jnp.float32),
            pltpu.SemaphoreType.DMA,
            pltpu.SemaphoreType.DMA,
        ],
        **_SC_PARAMS,
    )
    def passa1(qt_hbm, kt_hbm, src_hbm, dst_hbm, ea_hbm, zsd_hbm,
               p_out, sd_out,
               idx_s, idx_d, qrows, krows, earows, pbuf,
               sd_sp, sem0, sem1):
        cid = lax.axis_index("c")
        sid = lax.axis_index("s")
        wid = cid * 16 + sid
        r0 = sid * rows_per_tile
        pltpu.sync_copy(zsd_hbm.at[pl.ds(r0, rows_per_tile), :],
                        sd_sp.at[pl.ds(r0, rows_per_tile), :])
        plsc.subcore_barrier()

        lanes = lax.iota(jnp.int32, 16)

        def chunk_body(t, _):
            base = wid * per_tile + t * _CA
            pltpu.sync_copy(src_hbm.at[pl.ds(base, _CA)], idx_s)
            pltpu.sync_copy(dst_hbm.at[pl.ds(base, _CA)], idx_d)
            cp0 = pltpu.async_copy(qt_hbm.at[idx_d], qrows, sem0)
            cp1 = pltpu.async_copy(kt_hbm.at[idx_s], krows, sem1)
            pltpu.sync_copy(ea_hbm.at[pl.ds(base, _CA), :], earows)
            cp0.wait()
            cp1.wait()

            def edge_body(i, _):
                ea_v = earows[i, :]
                lv = jnp.full((16,), -1e9, jnp.float32)
                for h in range(H):
                    acc = (qrows[i, pl.ds(h * 128, 16)] *
                           krows[i, pl.ds(h * 128, 16)])
                    for t8 in range(1, 8):
                        acc = acc + (qrows[i, pl.ds(h * 128 + t8 * 16, 16)] *
                                     krows[i, pl.ds(h * 128 + t8 * 16, 16)])
                    acc = acc + qrows[i, pl.ds(512 + h * 16, 16)] * ea_v
                    lv = jnp.where(lanes == h, jnp.sum(acc) * SCALE, lv)
                gvec = (krows[i, pl.ds(512, 16)] + qrows[i, pl.ds(576, 16)])
                gvec = jnp.where(gvec > 0, gvec, 0.2 * gvec)
                for h in range(H):
                    lv = jnp.where(lanes == 4 + h, gvec[h], lv)
                lv = jnp.where(lanes == 8, 0.0, lv)
                pbuf[i, :] = jnp.exp(lv)
                return 0

            lax.fori_loop(0, _CA, edge_body, 0)
            pltpu.sync_copy(pbuf, p_out.at[pl.ds(base, _CA), :])
            pltpu.sync_copy(pbuf, sd_sp.at[idx_d], add=True)
            return 0

        lax.fori_loop(0, n_chunks, chunk_body, 0)
        plsc.subcore_barrier()
        pltpu.sync_copy(sd_sp.at[pl.ds(r0, rows_per_tile), :],
                        sd_out.at[cid, pl.ds(r0, rows_per_tile), :])

    return passa1(QT, KT, src, dst, ea, zero_sd)


_CS = 40


def _sc_pass_a2(X, src, dst, zero_agg):
    N = X.shape[0]
    NP = zero_agg.shape[0]
    E = src.shape[0]
    per_tile = E // 32
    n_chunks = per_tile // _CS
    rows_per_tile = NP // 16
    mesh = plsc.VectorSubcoreMesh(core_axis_name="c", subcore_axis_name="s")

    @functools.partial(
        pl.kernel, mesh=mesh,
        out_type=[jax.ShapeDtypeStruct((2, NP, 128), jnp.float32)],
        scratch_types=[
            pltpu.VMEM((_CS,), jnp.int32),
            pltpu.VMEM((_CS,), jnp.int32),
            pltpu.VMEM((_CS, 128), jnp.float32),
            pltpu.VMEM_SHARED((NP, 128), jnp.float32),
            pltpu.SemaphoreType.DMA,
        ],
        **_SC_PARAMS,
    )
    def passa2(x_hbm, src_hbm, dst_hbm, z_hbm, agg_out,
               idx_s, idx_d, xrows, agg_sp, sem0):
        cid = lax.axis_index("c")
        sid = lax.axis_index("s")
        wid = cid * 16 + sid
        r0 = sid * rows_per_tile
        pltpu.sync_copy(z_hbm.at[pl.ds(r0, rows_per_tile), :],
                        agg_sp.at[pl.ds(r0, rows_per_tile), :])
        plsc.subcore_barrier()

        def chunk_body(t, _):
            base = wid * per_tile + t * _CS
            pltpu.sync_copy(src_hbm.at[pl.ds(base, _CS)], idx_s)
            pltpu.sync_copy(dst_hbm.at[pl.ds(base, _CS)], idx_d)
            pltpu.async_copy(x_hbm.at[idx_s], xrows, sem0).wait()
            pltpu.sync_copy(xrows, agg_sp.at[idx_d], add=True)
            return 0

        lax.fori_loop(0, n_chunks, chunk_body, 0)
        plsc.subcore_barrier()
        pltpu.sync_copy(agg_sp.at[pl.ds(r0, rows_per_tile), :],
                        agg_out.at[cid, pl.ds(r0, rows_per_tile), :])

    return passa2(X, src, dst, zero_agg)


def _mid_body(agg0_ref, agg1_ref, sd0_ref, sd1_ref, asd_ref, x_ref, xh_ref,
              WlT_ref, bl_ref, WrT_ref,
              xs_ref, sinv_ref, st_ref):
    aggm = agg0_ref[...] + agg1_ref[...]
    sden = sd0_ref[...] + sd1_ref[...]
    cnt = jnp.clip(sden[:, 8:9], 1.0, None)
    xs = jnp.dot(aggm / cnt, WlT_ref[...],
                 preferred_element_type=jnp.float32) + bl_ref[...]
    xs = xs + jnp.dot(x_ref[...], WrT_ref[...], preferred_element_type=jnp.float32)
    xs_ref[...] = jnp.maximum(xs, 0.0)
    asd = asd_ref[...]
    a = asd[:, 0:H] + asd[:, H:2 * H]
    el = jnp.exp(jnp.where(a > 0, a, 0.2 * a))
    s = sden[:, 0:H]
    sg = sden[:, H:2 * H] + el
    sinv = 1.0 / (s + 1e-16)
    sginv = 1.0 / (sg + 1e-16)
    bn = sinv.shape[0]
    sinv_ref[...] = jnp.concatenate(
        [sinv, sginv, jnp.zeros((bn, 8), jnp.float32)], axis=1)
    coef = el * sginv
    xh = xh_ref[...]
    st = coef[:, 0:1] * xh[:, 0:128]
    for h in range(1, H):
        st = st + coef[:, h:h + 1] * xh[:, h * 128:(h + 1) * 128]
    st_ref[...] = st


def _tc_mid(agg0, agg1, sd0, sd1, asd, x, xh, WlT, bl, WrT):
    N, D = x.shape
    BN = 1000
    grid = (N // BN,)
    full = lambda shape: pl.BlockSpec(shape, lambda i: tuple(0 for _ in shape))
    row = lambda w: pl.BlockSpec((BN, w), lambda i: (i, 0))
    return pl.pallas_call(
        _mid_body,
        grid=grid,
        in_specs=[row(D), row(D), row(16), row(16), row(2 * H), row(D),
                  row(H * D), full((D, D)), full((1, D)), full((D, D))],
        out_specs=[row(D), row(16), row(D)],
        out_shape=[
            jax.ShapeDtypeStruct((N, D), jnp.float32),
            jax.ShapeDtypeStruct((N, 16), jnp.float32),
            jax.ShapeDtypeStruct((N, D), jnp.float32),
        ],
    )(agg0, agg1, sd0, sd1, asd, x, xh, WlT, bl, WrT)


_CB = 40


def _sc_pass_agg512(T512, SINV, P, src, dst, zero_out, lane_off):
    N = T512.shape[0]
    NP = zero_out.shape[0]
    E = src.shape[0]
    per_tile = E // 32
    n_chunks = per_tile // _CB
    rows_per_tile = NP // 16
    mesh = plsc.VectorSubcoreMesh(core_axis_name="c", subcore_axis_name="s")

    @functools.partial(
        pl.kernel, mesh=mesh,
        out_type=[jax.ShapeDtypeStruct((2, NP, 128), jnp.float32)],
        scratch_types=[
            pltpu.VMEM((_CB,), jnp.int32),
            pltpu.VMEM((_CB,), jnp.int32),
            pltpu.VMEM((_CB, 512), jnp.float32),
            pltpu.VMEM((_CB, 16), jnp.float32),
            pltpu.VMEM((_CB, 16), jnp.float32),
            pltpu.VMEM((_CB, 128), jnp.float32),
            pltpu.VMEM_SHARED((NP, 128), jnp.float32),
            pltpu.SemaphoreType.DMA,
            pltpu.SemaphoreType.DMA,
        ],
        **_SC_PARAMS,
    )
    def passagg(t_hbm, sinv_hbm, p_hbm, src_hbm, dst_hbm, z_hbm,
                out_hbm,
                idx_s, idx_d, trows, srows, prows, mbuf,
                acc_sp, sem0, sem1):
        cid = lax.axis_index("c")
        sid = lax.axis_index("s")
        wid = cid * 16 + sid
        r0 = sid * rows_per_tile
        pltpu.sync_copy(z_hbm.at[pl.ds(r0, rows_per_tile), :],
                        acc_sp.at[pl.ds(r0, rows_per_tile), :])
        plsc.subcore_barrier()

        def chunk_body(t, _):
            base = wid * per_tile + t * _CB
            pltpu.sync_copy(src_hbm.at[pl.ds(base, _CB)], idx_s)
            pltpu.sync_copy(dst_hbm.at[pl.ds(base, _CB)], idx_d)
            cp0 = pltpu.async_copy(t_hbm.at[idx_s], trows, sem0)
            cp1 = pltpu.async_copy(sinv_hbm.at[idx_d], srows, sem1)
            pltpu.sync_copy(p_hbm.at[pl.ds(base, _CB), :], prows)
            cp0.wait()
            cp1.wait()

            def edge_body(i, _):
                av = prows[i, :] * srows[i, :]
                a0 = av[lane_off]
                a1 = av[lane_off + 1]
                a2 = av[lane_off + 2]
                a3 = av[lane_off + 3]
                for t8 in range(8):
                    acc = a0 * trows[i, pl.ds(t8 * 16, 16)]
                    acc = acc + a1 * trows[i, pl.ds(128 + t8 * 16, 16)]
                    acc = acc + a2 * trows[i, pl.ds(256 + t8 * 16, 16)]
                    acc = acc + a3 * trows[i, pl.ds(384 + t8 * 16, 16)]
                    mbuf[i, pl.ds(t8 * 16, 16)] = acc
                return 0

            lax.fori_loop(0, _CB, edge_body, 0)
            pltpu.sync_copy(mbuf, acc_sp.at[idx_d], add=True)
            return 0

        lax.fori_loop(0, n_chunks, chunk_body, 0)
        plsc.subcore_barrier()
        pltpu.sync_copy(acc_sp.at[pl.ds(r0, rows_per_tile), :],
                        out_hbm.at[cid, pl.ds(r0, rows_per_tile), :])

    return passagg(T512, SINV, P, src, dst, zero_out)


_CR = 80


def _sc_pass_b2(SINV, P, src, dst, ea, zero_out):
    NP = zero_out.shape[0]
    E = src.shape[0]
    per_tile = E // 32
    n_chunks = per_tile // _CR
    rows_per_tile = NP // 16
    mesh = plsc.VectorSubcoreMesh(core_axis_name="c", subcore_axis_name="s")

    @functools.partial(
        pl.kernel, mesh=mesh,
        out_type=[jax.ShapeDtypeStruct((2, NP, 64), jnp.float32)],
        scratch_types=[
            pltpu.VMEM((_CR,), jnp.int32),
            pltpu.VMEM((_CR,), jnp.int32),
            pltpu.VMEM((_CR, 16), jnp.float32),
            pltpu.VMEM((_CR, 16), jnp.float32),
            pltpu.VMEM((_CR, 16), jnp.float32),
            pltpu.VMEM((_CR, 64), jnp.float32),
            pltpu.VMEM_SHARED((NP, 64), jnp.float32),
            pltpu.SemaphoreType.DMA,
        ],
        **_SC_PARAMS,
    )
    def passb2(sinv_hbm, p_hbm, src_hbm, dst_hbm, ea_hbm, z_hbm,
               out_hbm,
               idx_s, idx_d, srows, prows, earows, rbuf,
               acc_sp, sem0):
        cid = lax.axis_index("c")
        sid = lax.axis_index("s")
        wid = cid * 16 + sid
        r0 = sid * rows_per_tile
        pltpu.sync_copy(z_hbm.at[pl.ds(r0, rows_per_tile), :],
                        acc_sp.at[pl.ds(r0, rows_per_tile), :])
        plsc.subcore_barrier()

        def chunk_body(t, _):
            base = wid * per_tile + t * _CR
            pltpu.sync_copy(src_hbm.at[pl.ds(base, _CR)], idx_s)
            pltpu.sync_copy(dst_hbm.at[pl.ds(base, _CR)], idx_d)
            cp1 = pltpu.async_copy(sinv_hbm.at[idx_d], srows, sem0)
            pltpu.sync_copy(p_hbm.at[pl.ds(base, _CR), :], prows)
            pltpu.sync_copy(ea_hbm.at[pl.ds(base, _CR), :], earows)
            cp1.wait()

            def edge_body(i, _):
                av = prows[i, :] * srows[i, :]
                ea_v = earows[i, :]
                rbuf[i, pl.ds(0, 16)] = av[0] * ea_v
                rbuf[i, pl.ds(16, 16)] = av[1] * ea_v
                rbuf[i, pl.ds(32, 16)] = av[2] * ea_v
                rbuf[i, pl.ds(48, 16)] = av[3] * ea_v
                return 0

            lax.fori_loop(0, _CR, edge_body, 0)
            pltpu.sync_copy(rbuf, acc_sp.at[idx_d], add=True)
            return 0

        lax.fori_loop(0, n_chunks, chunk_body, 0)
        plsc.subcore_barrier()
        pltpu.sync_copy(acc_sp.at[pl.ds(r0, rows_per_tile), :],
                        out_hbm.at[cid, pl.ds(r0, rows_per_tile), :])

    return passb2(SINV, P, src, dst, ea, zero_out)


def _fin_body(x_ref, xs_ref, m0_ref, m1_ref, r0_ref, r1_ref,
              oc0_ref, oc1_ref, st_ref,
              W2_ref, WskipT_ref, bskip_ref, gatb_ref,
              F1T_ref, F2T_ref, F3T_ref, fb_ref,
              ln1w_ref, ln1b_ref, ln2w_ref, ln2b_ref,
              out_ref):
    x = x_ref[...]
    m = m0_ref[...] + m1_ref[...]
    r = r0_ref[...] + r1_ref[...]
    x_attn = (m + jnp.dot(r, W2_ref[...], preferred_element_type=jnp.float32)) * 0.25
    x_attn = x_attn + jnp.dot(x, WskipT_ref[...], preferred_element_type=jnp.float32)
    x_attn = jnp.maximum(x_attn + bskip_ref[...], 0.0)
    x_nb = jnp.maximum((oc0_ref[...] + oc1_ref[...] + st_ref[...]) * 0.25
                       + gatb_ref[...], 0.0)
    h = jnp.dot(xs_ref[...], F1T_ref[...], preferred_element_type=jnp.float32)
    h = h + jnp.dot(x_attn, F2T_ref[...], preferred_element_type=jnp.float32)
    h = h + jnp.dot(x_nb, F3T_ref[...], preferred_element_type=jnp.float32)
    h = h + fb_ref[...]
    mu = jnp.mean(h, axis=1, keepdims=True)
    var = jnp.mean((h - mu) * (h - mu), axis=1, keepdims=True)
    h = (h - mu) / jnp.sqrt(var + 1e-5) * ln1w_ref[...] + ln1b_ref[...]
    h = jnp.maximum(h, 0.0)
    h = x + h
    mu = jnp.mean(h, axis=1, keepdims=True)
    var = jnp.mean((h - mu) * (h - mu), axis=1, keepdims=True)
    out_ref[...] = (h - mu) / jnp.sqrt(var + 1e-5) * ln2w_ref[...] + ln2b_ref[...]


def _tc_fin(x, xs, m0, m1, r0, r1, oc0, oc1, st, W2, WskipT, bskip, gatb,
            F1T, F2T, F3T, fb, ln1w, ln1b, ln2w, ln2b):
    N, D = x.shape
    BN = 1000
    grid = (N // BN,)
    full = lambda shape: pl.BlockSpec(shape, lambda i: tuple(0 for _ in shape))
    row = lambda w: pl.BlockSpec((BN, w), lambda i: (i, 0))
    return pl.pallas_call(
        _fin_body,
        grid=grid,
        in_specs=[row(D), row(D), row(D), row(D), row(64), row(64),
                  row(D), row(D), row(D),
                  full((64, D)), full((D, D)), full((1, D)), full((1, D)),
                  full((D, D)), full((D, D)), full((D, D)), full((1, D)),
                  full((1, D)), full((1, D)), full((1, D)), full((1, D))],
        out_specs=[row(D)],
        out_shape=[jax.ShapeDtypeStruct((N, D), jnp.float32)],
    )(x, xs, m0, m1, r0, r1, oc0, oc1, st, W2, WskipT, bskip, gatb,
      F1T, F2T, F3T, fb, ln1w, ln1b, ln2w, ln2b)[0]


def kernel(x, edge_index, edge_attr, sage_Wl, sage_bl, sage_Wr, Wq, bq, Wk,
           bk, Wv, bv, We, be, Wskip, bskip, Wg, att_src, att_dst, gat_b,
           fuse_W, fuse_b, ln1_w, ln1_b, ln2_w, ln2_b, gate_s, gate_a, gate_n):
    N, D = x.shape
    E = edge_index.shape[1]
    src = edge_index[0]
    dst = edge_index[1]

    q, k, v, xh, qe, asd = _tc_prep(
        x, Wq.T, bq.reshape(1, -1), Wk.T, bk.reshape(1, -1), Wv.T,
        bv.reshape(1, -1), We, Wg.T, att_src, att_dst)
    k = k + be.reshape(1, -1)
    v = v + be.reshape(1, -1)

    pad12 = jnp.zeros((N, 12), jnp.float32)
    QT = jnp.concatenate([q, qe, asd[:, H:2 * H], pad12], axis=1)
    KT = jnp.concatenate([k, asd[:, 0:H], pad12], axis=1)

    NP = 16 * (-(-(N // 16) // 8) * 8)
    z16 = jnp.zeros((NP, 16), jnp.float32)
    z64 = jnp.zeros((NP, 64), jnp.float32)
    z128 = jnp.zeros((NP, 128), jnp.float32)

    P, sd_part = _sc_pass_a1(QT, KT, src, dst, edge_attr, z16)
    agg_part = _sc_pass_a2(x, src, dst, z128)[0][:, :N]
    sd_part = sd_part[:, :N]

    xs, SINV, st = _tc_mid(
        agg_part[0], agg_part[1], sd_part[0], sd_part[1], asd, x, xh,
        sage_Wl.T, sage_bl.reshape(1, -1), sage_Wr.T)

    outb = _sc_pass_agg512(v, SINV, P, src, dst, z128, 0)[0][:, :N]
    outr = _sc_pass_b2(SINV, P, src, dst, edge_attr, z64)[0][:, :N]
    outc = _sc_pass_agg512(xh, SINV, P, src, dst, z128, 4)[0][:, :N]

    W2 = We.reshape(H, D, ED).transpose(0, 2, 1).reshape(H * ED, D)
    sig = jax.nn.sigmoid
    F1T = fuse_W[:, 0:D].T * sig(gate_s)[0]
    F2T = fuse_W[:, D:2 * D].T * sig(gate_a)[0]
    F3T = fuse_W[:, 2 * D:3 * D].T * sig(gate_n)[0]

    return _tc_fin(
        x, xs, outb[0], outb[1], outr[0], outr[1], outc[0], outc[1], st,
        W2, Wskip.T, bskip.reshape(1, -1), gat_b.reshape(1, -1),
        F1T, F2T, F3T, fuse_b.reshape(1, -1),
        ln1_w.reshape(1, -1), ln1_b.reshape(1, -1),
        ln2_w.reshape(1, -1), ln2_b.reshape(1, -1))
